# Initial kernel scaffold; baseline (speedup 1.0000x reference)
import jax
import jax.numpy as jnp
from jax.experimental import pallas as pl


def kernel(x, adjs, weight, bias):
    raise NotImplementedError("write your pallas kernel here")


# speedup vs baseline: 1.0486x; 1.0486x over previous
import jax
import jax.numpy as jnp
from jax.experimental import pallas as pl


def _xw_kernel(x_ref, w_ref, o_ref):
    o_ref[...] = jnp.dot(
        x_ref[...].astype(jnp.bfloat16),
        w_ref[...].astype(jnp.bfloat16),
        preferred_element_type=jnp.float32,
    ).astype(jnp.bfloat16)


def _prop_kernel(a_ref, z_ref, o_ref):
    o_ref[...] = jnp.dot(
        a_ref[...].astype(jnp.bfloat16),
        z_ref[...],
        preferred_element_type=jnp.float32,
    ).astype(jnp.bfloat16)


def _prop_softmax_kernel(a_ref, y_ref, b_ref, o_ref):
    acc = jnp.dot(
        a_ref[...].astype(jnp.bfloat16),
        y_ref[...],
        preferred_element_type=jnp.float32,
    )
    v = acc + b_ref[...]
    m = jnp.max(v, axis=1, keepdims=True)
    lse = jnp.log(jnp.sum(jnp.exp(v - m), axis=1, keepdims=True)) + m
    o_ref[...] = v - lse


def kernel(x, adjs, weight, bias):
    n, d_in = x.shape
    d_out = weight.shape[1]
    a = adjs.reshape(n, n)
    bias2d = bias.reshape(1, d_out)

    z = pl.pallas_call(
        _xw_kernel,
        out_shape=jax.ShapeDtypeStruct((n, d_out), jnp.bfloat16),
    )(x, weight)

    bm = 400
    grid = (n // bm,)

    y = pl.pallas_call(
        _prop_kernel,
        grid=grid,
        in_specs=[
            pl.BlockSpec((bm, n), lambda i: (i, 0)),
            pl.BlockSpec((n, d_out), lambda i: (0, 0)),
        ],
        out_specs=pl.BlockSpec((bm, d_out), lambda i: (i, 0)),
        out_shape=jax.ShapeDtypeStruct((n, d_out), jnp.bfloat16),
    )(a, z)

    out = pl.pallas_call(
        _prop_softmax_kernel,
        grid=grid,
        in_specs=[
            pl.BlockSpec((bm, n), lambda i: (i, 0)),
            pl.BlockSpec((n, d_out), lambda i: (0, 0)),
            pl.BlockSpec((1, d_out), lambda i: (0, 0)),
        ],
        out_specs=pl.BlockSpec((bm, d_out), lambda i: (i, 0)),
        out_shape=jax.ShapeDtypeStruct((n, d_out), jnp.float32),
    )(a, y, bias2d)
    return out


# speedup vs baseline: 1.0841x; 1.0339x over previous
import jax
import jax.numpy as jnp
from jax.experimental import pallas as pl
from jax.experimental.pallas import tpu as pltpu

_BM = 400


def _fused_kernel(a_ref, x_ref, w_ref, b_ref, o_ref, z_ref, y_ref):
    pid = pl.program_id(0)
    nblk = pl.num_programs(0) // 2
    i = jax.lax.rem(pid, nblk)

    @pl.when(pid == 0)
    def _():
        z_ref[...] = jnp.dot(
            x_ref[...].astype(jnp.bfloat16),
            w_ref[...].astype(jnp.bfloat16),
            preferred_element_type=jnp.float32,
        ).astype(jnp.bfloat16)

    a_bf = a_ref[...].astype(jnp.bfloat16)

    @pl.when(pid < nblk)
    def _():
        y_ref[pl.ds(i * _BM, _BM), :] = jnp.dot(
            a_bf, z_ref[...], preferred_element_type=jnp.float32
        ).astype(jnp.bfloat16)

    @pl.when(pid >= nblk)
    def _():
        acc = jnp.dot(a_bf, y_ref[...], preferred_element_type=jnp.float32)
        v = acc + b_ref[...]
        m = jnp.max(v, axis=1, keepdims=True)
        lse = jnp.log(jnp.sum(jnp.exp(v - m), axis=1, keepdims=True)) + m
        o_ref[...] = v - lse


def kernel(x, adjs, weight, bias):
    n, d_in = x.shape
    d_out = weight.shape[1]
    a = adjs.reshape(n, n)
    bias2d = bias.reshape(1, d_out)
    nblk = n // _BM

    return pl.pallas_call(
        _fused_kernel,
        grid=(2 * nblk,),
        in_specs=[
            pl.BlockSpec((_BM, n), lambda p: (jax.lax.rem(p, n // _BM), 0)),
            pl.BlockSpec((n, d_in), lambda p: (0, 0)),
            pl.BlockSpec((d_in, d_out), lambda p: (0, 0)),
            pl.BlockSpec((1, d_out), lambda p: (0, 0)),
        ],
        out_specs=pl.BlockSpec((_BM, d_out), lambda p: (jax.lax.rem(p, n // _BM), 0)),
        out_shape=jax.ShapeDtypeStruct((n, d_out), jnp.float32),
        scratch_shapes=[
            pltpu.VMEM((n, d_out), jnp.bfloat16),
            pltpu.VMEM((n, d_out), jnp.bfloat16),
        ],
    )(a, x, weight, bias2d)
